# baseline (speedup 1.0000x reference)
import jax
import jax.numpy as jnp
from jax import lax
from jax.experimental import pallas as pl
from jax.experimental.pallas import tpu as pltpu
from jax.experimental.pallas import tpu_sc as plsc

H = 512
W = 512
N = H * W
MAXP = 2 * N
NB = 4
MAX_ITERS = 15
RPT = 32
BAND = RPT * W
HALO = BAND + 2 * W
NROW = BAND // 128
MAXROUNDS = 100


def _reorder(v):
    return jnp.minimum(jnp.where(v >= N, v - N, v), N - 1)


def _stencil(ah, vert, store_fn):
    lane = lax.iota(jnp.int32, 16)

    def vpass(i, _):
        o = i * 16
        a = ah[pl.ds(o, 16)]
        b = ah[pl.ds(o + W, 16)]
        c = ah[pl.ds(o + 2 * W, 16)]
        vert[pl.ds(o, 16)] = jnp.minimum(jnp.minimum(a, b), c)
        return 0

    lax.fori_loop(0, BAND // 16, vpass, 0)

    def hpass(i, _):
        o = i * 16
        col = o % W
        cv = vert[pl.ds(o, 16)]
        lv = vert[pl.ds(o - 1, 16)]
        rv = vert[pl.ds(o + 1, 16)]
        lv = jnp.where((col == 0) & (lane == 0), MAXP, lv)
        rv = jnp.where((col == W - 16) & (lane == 15), MAXP, rv)
        m3 = jnp.minimum(jnp.minimum(lv, cv), rv)
        center = ah[pl.ds(o + W, 16)]
        x2 = jnp.where(center < MAXP, m3, MAXP)
        store_fn(o, x2)
        return 0

    lax.fori_loop(0, BAND // 16, hpass, 0)


def _body(x_hbm, out_hbm, A, Bu, FL, ah, xb, t0, t1, t2, t3r,
          st16, flb, rf):
    cid = lax.axis_index("c")
    tid = lax.axis_index("s")
    rows0 = tid * RPT
    band0 = rows0 * W
    lane = lax.iota(jnp.int32, 16)
    dump = N + lane

    def exchange(val, reduce_min):
        st16[...] = jnp.full((16,), val, jnp.int32)
        pltpu.sync_copy(st16, FL.at[pl.ds(tid * 16, 16)])
        plsc.subcore_barrier()
        pltpu.sync_copy(FL, flb)

        def red(i, acc):
            v = flb[pl.ds(i * 16, 16)]
            return jnp.minimum(acc, v) if reduce_min else jnp.maximum(acc, v)

        acc0 = jnp.full((16,), MAXP if reduce_min else -MAXP, jnp.int32)
        acc = lax.fori_loop(0, 16, red, acc0)
        plsc.subcore_barrier()
        return jnp.min(acc) if reduce_min else jnp.max(acc)

    def fill16(i, _):
        o = i * 16
        xb[pl.ds(o, 16)] = jnp.zeros((16,), jnp.float32)
        xb[pl.ds((BAND + W) + o, 16)] = jnp.zeros((16,), jnp.float32)
        ah[pl.ds(o, 16)] = jnp.full((16,), MAXP, jnp.int32)
        ah[pl.ds((BAND + W) + o, 16)] = jnp.full((16,), MAXP, jnp.int32)
        return 0

    lax.fori_loop(0, W // 16, fill16, 0)

    for b in range(2):
        bb = cid * 2 + b

        def stage_x(r, _):
            ir = rows0 - 1 + r

            @pl.when((ir >= 0) & (ir < H))
            def _():
                pltpu.sync_copy(x_hbm.at[bb, 0, ir], xb.at[pl.ds(r * W, W)])

            return 0

        lax.fori_loop(0, 34, stage_x, 0)

        def vvchunk(i, _):
            o = i * 16
            xv = xb[pl.ds(o, 16)]
            pix = (rows0 - 1) * W + o + lane
            vv = jnp.where(xv == 0.0, MAXP,
                           jnp.where(xv > 1.0, N + pix, pix))
            ah[pl.ds(o, 16)] = vv
            return 0

        lax.fori_loop(0, HALO // 16, vvchunk, 0)

        def store_plain(o, vec):
            t1[pl.ds(o, 16)] = vec

        _stencil(ah, t2, store_plain)
        pltpu.sync_copy(t1, A.at[pl.ds(band0, BAND)])

        def iter_body(carry):
            it, done = carry
            plsc.subcore_barrier()

            @pl.when(tid == 0)
            def _():
                pltpu.sync_copy(A.at[pl.ds(0, 33 * W)], ah.at[pl.ds(W, 33 * W)])

            @pl.when(tid == 15)
            def _():
                pltpu.sync_copy(A.at[pl.ds(band0 - W, 33 * W)],
                                ah.at[pl.ds(0, 33 * W)])

            @pl.when((tid > 0) & (tid < 15))
            def _():
                pltpu.sync_copy(A.at[pl.ds(band0 - W, 34 * W)],
                                ah.at[pl.ds(0, 34 * W)])

            def store_idx(o, vec):
                t0[pl.ds(o, 16)] = _reorder(vec)

            _stencil(ah, t1, store_idx)

            def g1(j, _):
                pltpu.sync_copy(A.at[t0.at[pl.ds(j * 128, 128)]],
                                t1.at[pl.ds(j * 128, 128)])
                return 0

            lax.fori_loop(0, NROW, g1, 0)

            def comb(i, _):
                o = i * 16
                g = t1[pl.ds(o, 16)]
                c = ah[pl.ds(o + W, 16)]
                m = c < MAXP
                x1 = jnp.where(m, g, c)
                t1[pl.ds(o, 16)] = x1
                t2[pl.ds(o, 16)] = jnp.where(m, x1, MAXP)
                t0[pl.ds(o, 16)] = _reorder(c)
                return 0

            lax.fori_loop(0, BAND // 16, comb, 0)
            pltpu.sync_copy(t1, Bu.at[pl.ds(band0, BAND)])

            def initrf(j, _):
                rf[j] = 1
                return 0

            lax.fori_loop(0, NROW, initrf, 0)
            plsc.subcore_barrier()

            def hook_cond(c):
                pend, rnd = c
                return (pend > 0) & (rnd < MAXROUNDS)

            def hook_round(c):
                _, rnd = c

                def hrow(j, a):
                    rfj = rf[j]
                    rf[j] = 0

                    @pl.when(rfj > 0)
                    def _():
                        pltpu.sync_copy(Bu.at[t0.at[pl.ds(j * 128, 128)]],
                                        t3r)

                        def hvec(k, w):
                            o = j * 128 + k * 16
                            idxv = t0[pl.ds(o, 16)]
                            valv = t2[pl.ds(o, 16)]
                            curv = t3r[pl.ds(k * 16, 16)]
                            m = valv < curv
                            anym = jnp.any(m)

                            @pl.when(anym)
                            def _():
                                idxw = jnp.where(m, idxv, dump)
                                pltpu.sync_copy(t2.at[pl.ds(o, 16)],
                                                Bu.at[idxw])

                            return w | anym.astype(jnp.int32)

                        w = lax.fori_loop(0, 8, hvec, 0)
                        rf[j] = w

                    return a + rf[j]

                anyl = lax.fori_loop(0, NROW, hrow, 0)
                pend = exchange(anyl, reduce_min=False)
                return (pend, rnd + 1)

            lax.while_loop(hook_cond, hook_round, (1, 0))

            pltpu.sync_copy(Bu.at[pl.ds(band0, BAND)], t1)

            def mkidx(i, _):
                o = i * 16
                v = t1[pl.ds(o, 16)]
                t0[pl.ds(o, 16)] = _reorder(v)
                return 0

            lax.fori_loop(0, BAND // 16, mkidx, 0)

            def g2(j, _):
                pltpu.sync_copy(Bu.at[t0.at[pl.ds(j * 128, 128)]],
                                t2.at[pl.ds(j * 128, 128)])
                return 0

            lax.fori_loop(0, NROW, g2, 0)

            def comb2(i, cv):
                o = i * 16
                v = t1[pl.ds(o, 16)]
                g = t2[pl.ds(o, 16)]
                m = v < MAXP
                xn = jnp.where(m, g, v)
                t1[pl.ds(o, 16)] = xn
                c_old = ah[pl.ds(o + W, 16)]
                eq = jnp.all(xn == c_old)
                return cv & eq.astype(jnp.int32)

            convl = lax.fori_loop(0, BAND // 16, comb2, 1)
            pltpu.sync_copy(t1, A.at[pl.ds(band0, BAND)])
            convg = exchange(convl, reduce_min=True)
            return (it + 1, convg)

        lax.while_loop(lambda c: (c[0] < MAX_ITERS) & (c[1] == 0), iter_body,
                       (jnp.int32(0), jnp.int32(0)))

        plsc.subcore_barrier()

        def orow(r, _):
            pltpu.sync_copy(A.at[pl.ds((rows0 + r) * W, W)],
                            out_hbm.at[bb, 0, rows0 + r])
            return 0

        lax.fori_loop(0, RPT, orow, 0)
        plsc.subcore_barrier()


@jax.jit
def _run(x):
    mesh = plsc.VectorSubcoreMesh(core_axis_name="c", subcore_axis_name="s")
    f = pl.kernel(
        _body,
        out_type=jax.ShapeDtypeStruct((NB, 1, H, W), jnp.int32),
        mesh=mesh,
        compiler_params=pltpu.CompilerParams(needs_layout_passes=False),
        scratch_types=[
            pltpu.VMEM_SHARED((N,), jnp.int32),
            pltpu.VMEM_SHARED((N + 16,), jnp.int32),
            pltpu.VMEM_SHARED((256,), jnp.int32),
            pltpu.VMEM((HALO,), jnp.int32),
            pltpu.VMEM((HALO,), jnp.float32),
            pltpu.VMEM((BAND,), jnp.int32),
            pltpu.VMEM((BAND,), jnp.int32),
            pltpu.VMEM((BAND,), jnp.int32),
            pltpu.VMEM((128,), jnp.int32),
            pltpu.VMEM((16,), jnp.int32),
            pltpu.VMEM((256,), jnp.int32),
            pltpu.SMEM((NROW,), jnp.int32),
        ],
    )
    return f(x)


def kernel(x):
    return _run(x)
